# baseline (speedup 1.0000x reference)
import functools

import jax
import jax.numpy as jnp
from jax import lax
from jax.experimental import pallas as pl
from jax.experimental.pallas import tpu as pltpu
from jax.experimental.pallas import tpu_sc as plsc

NQ = 1024
D = 128
NV = 100000
NPAD = 100352
NCHUNK = NPAD // 128
BN = 2048
NBLK = NPAD // BN
BC = BN // 128
KTOP = 100
CAND = 256
CHK = 128
NEG = -3.0e38
BIG = 1 << 30


def _mm_body(q_ref, v_ref, s_ref, cm_ref):
    i = pl.program_id(0)
    s = lax.dot_general(q_ref[...], v_ref[...], (((1,), (1,)), ((), ())))

    @pl.when(i < NBLK - 1)
    def _():
        s_ref[...] = s
        cm_ref[...] = jnp.max(s.reshape(NQ, BC, 128), axis=2)[None]

    @pl.when(i == NBLK - 1)
    def _():
        cols = (NBLK - 1) * BN + lax.broadcasted_iota(jnp.int32, (NQ, BN), 1)
        sm = jnp.where(cols < NV, s, NEG)
        s_ref[...] = sm
        cm_ref[...] = jnp.max(sm.reshape(NQ, BC, 128), axis=2)[None]


def _scores_and_chunkmax(query, vpad):
    return pl.pallas_call(
        _mm_body,
        grid=(NBLK,),
        in_specs=[
            pl.BlockSpec((NQ, D), lambda i: (0, 0)),
            pl.BlockSpec((BN, D), lambda i: (i, 0)),
        ],
        out_specs=[
            pl.BlockSpec((NQ, BN), lambda i: (0, i)),
            pl.BlockSpec((1, NQ, BC), lambda i: (i, 0, 0)),
        ],
        out_shape=[
            jax.ShapeDtypeStruct((NQ, NPAD), jnp.float32),
            jax.ShapeDtypeStruct((NBLK, NQ, BC), jnp.float32),
        ],
    )(query, vpad)


def _thresh_body(cm_ref, t_ref):
    cm = cm_ref[...]
    ub = lax.bitcast_convert_type(cm, jnp.uint32)
    top = jnp.uint32(0x80000000)
    ukey = jnp.where(ub >= top, ~ub, ub | top)

    def body(b, t_u):
        bit = jnp.uint32(1) << (31 - b)
        cand = t_u | bit
        cnt = jnp.sum((ukey >= cand).astype(jnp.int32), axis=1, keepdims=True)
        return jnp.where(cnt >= KTOP, cand, t_u)

    t_u = lax.fori_loop(0, 32, body, jnp.zeros((NQ, 1), jnp.uint32))
    fb = jnp.where(t_u >= top, t_u & jnp.uint32(0x7FFFFFFF), ~t_u)
    t_f = lax.bitcast_convert_type(fb, jnp.float32)
    t_ref[...] = jnp.broadcast_to(t_f, (NQ, 16))


def _threshold(cmax):
    return pl.pallas_call(
        _thresh_body,
        in_specs=[pl.BlockSpec((NQ, NCHUNK), lambda: (0, 0))],
        out_specs=pl.BlockSpec((NQ, 16), lambda: (0, 0)),
        out_shape=jax.ShapeDtypeStruct((NQ, 16), jnp.float32),
    )(cmax)


def _sc_select(scores2d, cmax, tb):
    mesh = plsc.VectorSubcoreMesh(core_axis_name="c", subcore_axis_name="s")
    rows_per_w = NQ // 32

    @functools.partial(
        pl.kernel,
        mesh=mesh,
        compiler_params=pltpu.CompilerParams(needs_layout_passes=False),
        out_type=[
            jax.ShapeDtypeStruct((NQ, CAND), jnp.float32),
            jax.ShapeDtypeStruct((NQ, CAND), jnp.int32),
        ],
        scratch_types=[
            pltpu.VMEM((NCHUNK,), jnp.float32),
            pltpu.VMEM((16,), jnp.float32),
            pltpu.VMEM((CHK,), jnp.int32),
            pltpu.VMEM((CHK + 16,), jnp.int32),
            pltpu.VMEM((CHK, 128), jnp.float32),
            pltpu.VMEM((CAND,), jnp.float32),
            pltpu.VMEM((CAND,), jnp.int32),
            pltpu.SemaphoreType.DMA,
        ],
    )
    def k3(scores_hbm, cmax_hbm, tb_hbm, cv_hbm, ci_hbm,
           cmax_v, tb_v, ids_v, ids_p, gath_v, candv_v, candi_v, sem):
        wid = lax.axis_index("s") * 2 + lax.axis_index("c")

        def row_body(r, carry):
            row = wid * rows_per_w + r
            base = row * NCHUNK
            pltpu.sync_copy(cmax_hbm.at[row], cmax_v)
            pltpu.sync_copy(tb_hbm.at[row], tb_v)
            t = tb_v[...]

            for j in range(CHK // 16):
                ids_v[pl.ds(j * 16, 16)] = jnp.zeros((16,), jnp.int32) + base
            for j in range(CHK // 16 + 1):
                ids_p[pl.ds(j * 16, 16)] = jnp.zeros((16,), jnp.int32) + base

            def cbody(j, off):
                v = cmax_v[pl.ds(j * 16, 16)]
                msk = v >= t
                cnt = jnp.sum(msk.astype(jnp.int32))
                gids = base + j * 16 + lax.iota(jnp.int32, 16)

                @pl.when((cnt > 0) & (off <= CHK - 16))
                def _():
                    plsc.store_compressed(ids_v.at[pl.ds(off, 16)], gids, mask=msk)
                    plsc.store_compressed(ids_p.at[pl.ds(off, 16)], gids, mask=msk)

                return jnp.minimum(off + cnt, CHK)

            n_sel = lax.fori_loop(0, NCHUNK // 16, cbody, 0)

            pltpu.async_copy(scores_hbm.at[ids_v], gath_v, sem).wait()

            for j in range(CAND // 16):
                candv_v[pl.ds(j * 16, 16)] = jnp.full((16,), NEG, jnp.float32)
                candi_v[pl.ds(j * 16, 16)] = jnp.zeros((16,), jnp.int32)

            def sbody(j2, off2):
                c_g = ids_p[pl.ds(j2, 16)][0]
                col0 = (c_g - base) * 128
                vs, msks, cnts = [], [], []
                for s in range(8):
                    v = gath_v[j2, pl.ds(s * 16, 16)]
                    msk = v >= t
                    vs.append(v)
                    msks.append(msk)
                    cnts.append(jnp.sum(msk.astype(jnp.int32)))
                offs, o = [], off2
                for s in range(8):
                    offs.append(o)
                    o = o + cnts[s]
                for s in range(8):
                    off_ok = jnp.minimum(offs[s], CAND - 16)
                    cols = col0 + s * 16 + lax.iota(jnp.int32, 16)

                    @pl.when(cnts[s] > 0)
                    def _(off_ok=off_ok, s=s, cols=cols):
                        plsc.store_compressed(
                            candv_v.at[pl.ds(off_ok, 16)], vs[s], mask=msks[s])
                        plsc.store_compressed(
                            candi_v.at[pl.ds(off_ok, 16)], cols, mask=msks[s])

                return jnp.minimum(o, CAND)

            lax.fori_loop(0, n_sel, sbody, 0)

            pltpu.sync_copy(candv_v, cv_hbm.at[row])
            pltpu.sync_copy(candi_v, ci_hbm.at[row])
            return carry

        lax.fori_loop(0, rows_per_w, row_body, 0)

    return k3(scores2d, cmax, tb)


def _topk_body(cv_ref, ci_ref, os_ref, oi_ref, work):
    work[...] = cv_ref[...]
    idx = ci_ref[...]
    lane = lax.broadcasted_iota(jnp.int32, (NQ, 128), 1)
    os_ref[...] = jnp.zeros((NQ, 128), jnp.float32)
    oi_ref[...] = jnp.zeros((NQ, 128), jnp.int32)

    def body(it, carry):
        v = work[...]
        m = jnp.max(v, axis=1, keepdims=True)
        is_m = v == m
        sel = jnp.min(jnp.where(is_m, idx, BIG), axis=1, keepdims=True)
        hit = lane == it
        os_ref[...] = jnp.where(hit, m, os_ref[...])
        oi_ref[...] = jnp.where(hit, sel, oi_ref[...])
        work[...] = jnp.where(is_m & (idx == sel), NEG, v)
        return carry

    lax.fori_loop(0, KTOP, body, 0)


def _topk(cv, ci):
    return pl.pallas_call(
        _topk_body,
        in_specs=[
            pl.BlockSpec((NQ, CAND), lambda: (0, 0)),
            pl.BlockSpec((NQ, CAND), lambda: (0, 0)),
        ],
        out_specs=[
            pl.BlockSpec((NQ, 128), lambda: (0, 0)),
            pl.BlockSpec((NQ, 128), lambda: (0, 0)),
        ],
        out_shape=[
            jax.ShapeDtypeStruct((NQ, 128), jnp.float32),
            jax.ShapeDtypeStruct((NQ, 128), jnp.int32),
        ],
        scratch_shapes=[pltpu.VMEM((NQ, CAND), jnp.float32)],
    )(cv, ci)


def kernel(query, vectors, k):
    del k
    vpad = jnp.concatenate(
        [vectors, jnp.zeros((NPAD - NV, D), jnp.float32)], axis=0)
    scores, cmax3 = _scores_and_chunkmax(query, vpad)
    cmax = cmax3.transpose(1, 0, 2).reshape(NQ, NCHUNK)
    tb = _threshold(cmax)
    scores2d = scores.reshape(NQ * NCHUNK, 128)
    cv, ci = _sc_select(scores2d, cmax, tb)
    os_, oi_ = _topk(cv, ci)
    return (os_[:, :KTOP], oi_[:, :KTOP])


# speedup vs baseline: 1.1679x; 1.1679x over previous
import functools

import jax
import jax.numpy as jnp
from jax import lax
from jax.experimental import pallas as pl
from jax.experimental.pallas import tpu as pltpu
from jax.experimental.pallas import tpu_sc as plsc

NQ = 1024
D = 128
NV = 100000
NPAD = 100352
NCHUNK = NPAD // 128
BN = 2048
NBLK = NPAD // BN
BC = BN // 128
KTOP = 100
CAND = 256
CHK = 128
NEG = -3.0e38
BIG = 1 << 30


def _mm_body(q_ref, v_ref, s_ref, cm_ref):
    i = pl.program_id(0)
    s = lax.dot_general(q_ref[...], v_ref[...], (((1,), (1,)), ((), ())))
    cols = i * BN + lax.broadcasted_iota(jnp.int32, (NQ, BN), 1)
    s = jnp.where(cols < NV, s, NEG)
    s_ref[...] = s
    cm_ref[...] = jnp.max(s.reshape(NQ, BC, 128), axis=2)[None]


def _scores_and_chunkmax(query, vpad):
    return pl.pallas_call(
        _mm_body,
        grid=(NBLK,),
        in_specs=[
            pl.BlockSpec((NQ, D), lambda i: (0, 0)),
            pl.BlockSpec((BN, D), lambda i: (i, 0)),
        ],
        out_specs=[
            pl.BlockSpec((NQ, BN), lambda i: (0, i)),
            pl.BlockSpec((1, NQ, BC), lambda i: (i, 0, 0)),
        ],
        out_shape=[
            jax.ShapeDtypeStruct((NQ, NPAD), jnp.float32),
            jax.ShapeDtypeStruct((NBLK, NQ, BC), jnp.float32),
        ],
    )(query, vpad)


def _thresh_body(cm_ref, t_ref):
    cm = cm_ref[...]
    ub = lax.bitcast_convert_type(cm, jnp.uint32)
    top = jnp.uint32(0x80000000)
    ukey = jnp.where(ub >= top, ~ub, ub | top)

    def body(b, t_u):
        bit = jnp.uint32(1) << (31 - b)
        cand = t_u | bit
        cnt = jnp.sum((ukey >= cand).astype(jnp.int32), axis=1, keepdims=True)
        return jnp.where(cnt >= KTOP, cand, t_u)

    t_u = lax.fori_loop(0, 32, body, jnp.zeros((NQ, 1), jnp.uint32))
    fb = jnp.where(t_u >= top, t_u & jnp.uint32(0x7FFFFFFF), ~t_u)
    t_f = lax.bitcast_convert_type(fb, jnp.float32)
    t_ref[...] = jnp.broadcast_to(t_f, (NQ, 16))


def _threshold(cmax):
    return pl.pallas_call(
        _thresh_body,
        in_specs=[pl.BlockSpec((NQ, NCHUNK), lambda: (0, 0))],
        out_specs=pl.BlockSpec((NQ, 16), lambda: (0, 0)),
        out_shape=jax.ShapeDtypeStruct((NQ, 16), jnp.float32),
    )(cmax)


def _sc_select(scores2d, cmax, tb):
    mesh = plsc.VectorSubcoreMesh(core_axis_name="c", subcore_axis_name="s")
    rows_per_w = NQ // 32
    npairs = rows_per_w // 2

    @functools.partial(
        pl.kernel,
        mesh=mesh,
        compiler_params=pltpu.CompilerParams(needs_layout_passes=False),
        out_type=[
            jax.ShapeDtypeStruct((NQ, CAND), jnp.float32),
            jax.ShapeDtypeStruct((NQ, CAND), jnp.int32),
        ],
        scratch_types=[
            [pltpu.VMEM((NCHUNK,), jnp.float32)] * 2,
            [pltpu.VMEM((16,), jnp.float32)] * 2,
            [pltpu.VMEM((16,), jnp.float32)] * 2,
            [pltpu.VMEM((CHK,), jnp.int32)] * 2,
            [pltpu.VMEM((CHK + 16,), jnp.int32)] * 2,
            [pltpu.VMEM((CHK, 128), jnp.float32)] * 2,
            [pltpu.VMEM((CAND,), jnp.float32)] * 2,
            [pltpu.VMEM((CAND,), jnp.int32)] * 2,
            [pltpu.SemaphoreType.DMA] * 2,
            [pltpu.SemaphoreType.DMA] * 2,
            [pltpu.SemaphoreType.DMA] * 2,
        ],
    )
    def k3(scores_hbm, cmax_hbm, tb_hbm, cv_hbm, ci_hbm,
           cmax_v, tb_v, tsv_v, ids_v, ids_p, gath_v, candv_v, candi_v,
           sem_c, sem_g, sem_o):
        wid = lax.axis_index("s") * 2 + lax.axis_index("c")
        row0 = wid * rows_per_w

        def loads_start(row, b):
            pltpu.async_copy(cmax_hbm.at[row], cmax_v[b], sem_c[b])
            pltpu.async_copy(tb_hbm.at[row], tb_v[b], sem_c[b])

        def loads_wait(b):
            pltpu.make_async_copy(cmax_hbm.at[0], cmax_v[b], sem_c[b]).wait()
            pltpu.make_async_copy(tb_hbm.at[0], tb_v[b], sem_c[b]).wait()

        def compact(row, b):
            base = row * NCHUNK
            tsv_v[b][...] = tb_v[b][...]
            t = tsv_v[b][...]
            for j in range(CHK // 16):
                ids_v[b][pl.ds(j * 16, 16)] = jnp.zeros((16,), jnp.int32) + base
            for j in range(CHK // 16 + 1):
                ids_p[b][pl.ds(j * 16, 16)] = jnp.zeros((16,), jnp.int32) + base

            def cbody(j, off):
                v = cmax_v[b][pl.ds(j * 16, 16)]
                msk = v >= t
                cnt = jnp.sum(msk.astype(jnp.int32))
                gids = base + j * 16 + lax.iota(jnp.int32, 16)

                @pl.when((cnt > 0) & (off <= CHK - 16))
                def _():
                    plsc.store_compressed(
                        ids_v[b].at[pl.ds(off, 16)], gids, mask=msk)
                    plsc.store_compressed(
                        ids_p[b].at[pl.ds(off, 16)], gids, mask=msk)

                return jnp.minimum(off + cnt, CHK)

            return lax.fori_loop(0, NCHUNK // 16, cbody, 0)

        def gather_start(b):
            pltpu.async_copy(scores_hbm.at[ids_v[b]], gath_v[b], sem_g[b])

        def gather_wait(b):
            pltpu.make_async_copy(
                scores_hbm.at[ids_v[b]], gath_v[b], sem_g[b]).wait()

        def out_wait(b):
            pltpu.make_async_copy(candv_v[b], cv_hbm.at[0], sem_o[b]).wait()
            pltpu.make_async_copy(candi_v[b], ci_hbm.at[0], sem_o[b]).wait()

        def scan_row(row, n_sel, b):
            base = row * NCHUNK
            t = tsv_v[b][...]
            for j in range(CAND // 16):
                candv_v[b][pl.ds(j * 16, 16)] = jnp.full((16,), NEG,
                                                         jnp.float32)
                candi_v[b][pl.ds(j * 16, 16)] = jnp.zeros((16,), jnp.int32)

            def sbody(j2, off2):
                c_g = ids_p[b][pl.ds(j2, 16)][0]
                col0 = (c_g - base) * 128
                vs, msks, cnts = [], [], []
                for s in range(8):
                    v = gath_v[b][j2, pl.ds(s * 16, 16)]
                    msk = v >= t
                    vs.append(v)
                    msks.append(msk)
                    cnts.append(jnp.sum(msk.astype(jnp.int32)))
                offs, o = [], off2
                for s in range(8):
                    offs.append(o)
                    o = o + cnts[s]
                for s in range(8):
                    off_ok = jnp.minimum(offs[s], CAND - 16)
                    cols = col0 + s * 16 + lax.iota(jnp.int32, 16)

                    @pl.when(cnts[s] > 0)
                    def _(off_ok=off_ok, s=s, cols=cols):
                        plsc.store_compressed(
                            candv_v[b].at[pl.ds(off_ok, 16)], vs[s],
                            mask=msks[s])
                        plsc.store_compressed(
                            candi_v[b].at[pl.ds(off_ok, 16)], cols,
                            mask=msks[s])

                return jnp.minimum(o, CAND)

            lax.fori_loop(0, n_sel, sbody, 0)
            pltpu.async_copy(candv_v[b], cv_hbm.at[row], sem_o[b])
            pltpu.async_copy(candi_v[b], ci_hbm.at[row], sem_o[b])

        loads_start(row0, 0)
        loads_start(row0 + 1, 1)

        def pair_body(g, n_sel_b):
            a = row0 + 2 * g
            loads_wait(0)
            n_sel_a = compact(a, 0)

            @pl.when(g < npairs - 1)
            def _():
                loads_start(a + 2, 0)

            gather_start(0)

            @pl.when(g > 1)
            def _():
                out_wait(1)

            @pl.when(g > 0)
            def _():
                gather_wait(1)
                scan_row(a - 1, n_sel_b, 1)

            loads_wait(1)
            n_sel_b2 = compact(a + 1, 1)

            @pl.when(g < npairs - 1)
            def _():
                loads_start(a + 3, 1)

            gather_start(1)

            gather_wait(0)

            @pl.when(g > 0)
            def _():
                out_wait(0)

            scan_row(a, n_sel_a, 0)
            return n_sel_b2

        n_sel_last = lax.fori_loop(0, npairs, pair_body, 0)
        gather_wait(1)
        out_wait(1)
        scan_row(row0 + rows_per_w - 1, n_sel_last, 1)
        out_wait(0)
        out_wait(1)

    return k3(scores2d, cmax, tb)


def _topk_body(cv_ref, ci_ref, os_ref, oi_ref, work):
    work[...] = cv_ref[...]
    idx = ci_ref[...]
    lane = lax.broadcasted_iota(jnp.int32, (NQ, 128), 1)
    os_ref[...] = jnp.zeros((NQ, 128), jnp.float32)
    oi_ref[...] = jnp.zeros((NQ, 128), jnp.int32)

    def body(it, carry):
        v = work[...]
        m = jnp.max(v, axis=1, keepdims=True)
        is_m = v == m
        sel = jnp.min(jnp.where(is_m, idx, BIG), axis=1, keepdims=True)
        hit = lane == it
        os_ref[...] = jnp.where(hit, m, os_ref[...])
        oi_ref[...] = jnp.where(hit, sel, oi_ref[...])
        work[...] = jnp.where(is_m & (idx == sel), NEG, v)
        return carry

    lax.fori_loop(0, KTOP, body, 0)


def _topk(cv, ci):
    return pl.pallas_call(
        _topk_body,
        in_specs=[
            pl.BlockSpec((NQ, CAND), lambda: (0, 0)),
            pl.BlockSpec((NQ, CAND), lambda: (0, 0)),
        ],
        out_specs=[
            pl.BlockSpec((NQ, 128), lambda: (0, 0)),
            pl.BlockSpec((NQ, 128), lambda: (0, 0)),
        ],
        out_shape=[
            jax.ShapeDtypeStruct((NQ, 128), jnp.float32),
            jax.ShapeDtypeStruct((NQ, 128), jnp.int32),
        ],
        scratch_shapes=[pltpu.VMEM((NQ, CAND), jnp.float32)],
    )(cv, ci)


def kernel(query, vectors, k):
    del k
    vpad = jnp.concatenate(
        [vectors, jnp.zeros((NPAD - NV, D), jnp.float32)], axis=0)
    scores, cmax3 = _scores_and_chunkmax(query, vpad)
    cmax = cmax3.transpose(1, 0, 2).reshape(NQ, NCHUNK)
    tb = _threshold(cmax)
    scores2d = scores.reshape(NQ * NCHUNK, 128)
    cv, ci = _sc_select(scores2d, cmax, tb)
    os_, oi_ = _topk(cv, ci)
    return (os_[:, :KTOP], oi_[:, :KTOP])


# speedup vs baseline: 1.2046x; 1.0314x over previous
import functools

import jax
import jax.numpy as jnp
from jax import lax
from jax.experimental import pallas as pl
from jax.experimental.pallas import tpu as pltpu
from jax.experimental.pallas import tpu_sc as plsc

NQ = 1024
D = 128
NV = 100000
NPAD = 100352
NCHUNK = NPAD // 128
BN = 2048
NBLK = NPAD // BN
BC = BN // 128
KTOP = 100
CAND = 256
CHK = 128
NEG = -3.0e38
BIG = 1 << 30


def _mm_body(q_ref, v_ref, s_ref, cm_ref):
    i = pl.program_id(0)
    s = lax.dot_general(q_ref[...], v_ref[...], (((1,), (1,)), ((), ())))
    cols = i * BN + lax.broadcasted_iota(jnp.int32, (NQ, BN), 1)
    s = jnp.where(cols < NV, s, NEG)
    s_ref[...] = s
    cm_ref[...] = jnp.max(s.reshape(NQ, BC, 128), axis=2)[None]


def _scores_and_chunkmax(query, vpad):
    return pl.pallas_call(
        _mm_body,
        grid=(NBLK,),
        in_specs=[
            pl.BlockSpec((NQ, D), lambda i: (0, 0)),
            pl.BlockSpec((BN, D), lambda i: (i, 0)),
        ],
        out_specs=[
            pl.BlockSpec((NQ, BN), lambda i: (0, i)),
            pl.BlockSpec((1, NQ, BC), lambda i: (i, 0, 0)),
        ],
        out_shape=[
            jax.ShapeDtypeStruct((NQ, NPAD), jnp.float32),
            jax.ShapeDtypeStruct((NBLK, NQ, BC), jnp.float32),
        ],
    )(query, vpad)


def _thresh_body(cm_ref, t_ref):
    cm = cm_ref[...]
    ub = lax.bitcast_convert_type(cm, jnp.uint32)
    top = jnp.uint32(0x80000000)
    ukey = jnp.where(ub >= top, ~ub, ub | top)

    def body(b, t_u):
        bit = jnp.uint32(1) << (31 - b)
        cand = t_u | bit
        cnt = jnp.sum((ukey >= cand).astype(jnp.int32), axis=1, keepdims=True)
        return jnp.where(cnt >= KTOP, cand, t_u)

    t_u = lax.fori_loop(0, 32, body, jnp.zeros((NQ, 1), jnp.uint32))
    fb = jnp.where(t_u >= top, t_u & jnp.uint32(0x7FFFFFFF), ~t_u)
    t_f = lax.bitcast_convert_type(fb, jnp.float32)
    t_ref[...] = jnp.broadcast_to(t_f, (NQ, 16))


def _threshold(cmax):
    return pl.pallas_call(
        _thresh_body,
        in_specs=[pl.BlockSpec((NQ, NCHUNK), lambda: (0, 0))],
        out_specs=pl.BlockSpec((NQ, 16), lambda: (0, 0)),
        out_shape=jax.ShapeDtypeStruct((NQ, 16), jnp.float32),
    )(cmax)


def _sc_select(scores2d, cmax, tb):
    mesh = plsc.VectorSubcoreMesh(core_axis_name="c", subcore_axis_name="s")
    rows_per_w = NQ // 32
    npairs = rows_per_w // 2

    @functools.partial(
        pl.kernel,
        mesh=mesh,
        compiler_params=pltpu.CompilerParams(needs_layout_passes=False),
        out_type=[
            jax.ShapeDtypeStruct((NQ, CAND), jnp.float32),
            jax.ShapeDtypeStruct((NQ, CAND), jnp.int32),
        ],
        scratch_types=[
            [pltpu.VMEM((NCHUNK,), jnp.float32)] * 2,
            [pltpu.VMEM((16,), jnp.float32)] * 2,
            [pltpu.VMEM((16,), jnp.float32)] * 2,
            [pltpu.VMEM((CHK,), jnp.int32)] * 2,
            [pltpu.VMEM((CHK + 16,), jnp.int32)] * 2,
            [pltpu.VMEM((CHK, 128), jnp.float32)] * 2,
            [pltpu.VMEM((CAND,), jnp.float32)] * 2,
            [pltpu.VMEM((CAND,), jnp.int32)] * 2,
            [pltpu.SemaphoreType.DMA] * 2,
            [pltpu.SemaphoreType.DMA] * 2,
            [pltpu.SemaphoreType.DMA] * 2,
        ],
    )
    def k3(scores_hbm, cmax_hbm, tb_hbm, cv_hbm, ci_hbm,
           cmax_v, tb_v, tsv_v, ids_v, ids_p, gath_v, candv_v, candi_v,
           sem_c, sem_g, sem_o):
        wid = lax.axis_index("s") * 2 + lax.axis_index("c")
        row0 = wid * rows_per_w

        def loads_start(row, b):
            pltpu.async_copy(cmax_hbm.at[row], cmax_v[b], sem_c[b])
            pltpu.async_copy(tb_hbm.at[row], tb_v[b], sem_c[b])

        def loads_wait(b):
            pltpu.make_async_copy(cmax_hbm.at[0], cmax_v[b], sem_c[b]).wait()
            pltpu.make_async_copy(tb_hbm.at[0], tb_v[b], sem_c[b]).wait()

        def compact(row, b):
            base = row * NCHUNK
            tsv_v[b][...] = tb_v[b][...]
            t = tsv_v[b][...]
            for j in range(CHK // 16):
                ids_v[b][pl.ds(j * 16, 16)] = jnp.zeros((16,), jnp.int32) + base
            for j in range(CHK // 16 + 1):
                ids_p[b][pl.ds(j * 16, 16)] = jnp.zeros((16,), jnp.int32) + base

            def cbody(u, off):
                msks, cnts = [], []
                for q in range(7):
                    v = cmax_v[b][pl.ds((u * 7 + q) * 16, 16)]
                    msk = v >= t
                    msks.append(msk)
                    cnts.append(jnp.sum(msk.astype(jnp.int32)))
                offs, o = [], off
                for q in range(7):
                    offs.append(o)
                    o = o + cnts[q]
                for q in range(7):
                    off_ok = jnp.minimum(offs[q], CHK - 16)
                    gids = base + (u * 7 + q) * 16 + lax.iota(jnp.int32, 16)

                    @pl.when(cnts[q] > 0)
                    def _(off_ok=off_ok, gids=gids, q=q):
                        plsc.store_compressed(
                            ids_v[b].at[pl.ds(off_ok, 16)], gids, mask=msks[q])
                        plsc.store_compressed(
                            ids_p[b].at[pl.ds(off_ok, 16)], gids, mask=msks[q])

                return jnp.minimum(o, CHK)

            return lax.fori_loop(0, NCHUNK // 16 // 7, cbody, 0)

        def gather_start(b):
            pltpu.async_copy(scores_hbm.at[ids_v[b]], gath_v[b], sem_g[b])

        def gather_wait(b):
            pltpu.make_async_copy(
                scores_hbm.at[ids_v[b]], gath_v[b], sem_g[b]).wait()

        def out_wait(b):
            pltpu.make_async_copy(candv_v[b], cv_hbm.at[0], sem_o[b]).wait()
            pltpu.make_async_copy(candi_v[b], ci_hbm.at[0], sem_o[b]).wait()

        def scan_row(row, n_sel, b):
            base = row * NCHUNK
            t = tsv_v[b][...]
            for j in range(CAND // 16):
                candv_v[b][pl.ds(j * 16, 16)] = jnp.full((16,), NEG,
                                                         jnp.float32)
                candi_v[b][pl.ds(j * 16, 16)] = jnp.zeros((16,), jnp.int32)

            def scan_chunk(j2, off2):
                c_g = ids_p[b][pl.ds(j2, 16)][0]
                col0 = (c_g - base) * 128
                vs, msks, cnts = [], [], []
                for s in range(8):
                    v = gath_v[b][j2, pl.ds(s * 16, 16)]
                    msk = v >= t
                    vs.append(v)
                    msks.append(msk)
                    cnts.append(jnp.sum(msk.astype(jnp.int32)))
                offs, o = [], off2
                for s in range(8):
                    offs.append(o)
                    o = o + cnts[s]
                for s in range(8):
                    off_ok = jnp.minimum(offs[s], CAND - 16)
                    cols = col0 + s * 16 + lax.iota(jnp.int32, 16)

                    @pl.when(cnts[s] > 0)
                    def _(off_ok=off_ok, s=s, cols=cols):
                        plsc.store_compressed(
                            candv_v[b].at[pl.ds(off_ok, 16)], vs[s],
                            mask=msks[s])
                        plsc.store_compressed(
                            candi_v[b].at[pl.ds(off_ok, 16)], cols,
                            mask=msks[s])

                return jnp.minimum(o, CAND)

            def sbody(u, off2):
                off2 = scan_chunk(2 * u, off2)
                return scan_chunk(2 * u + 1, off2)

            off_fin = lax.fori_loop(0, n_sel // 2, sbody, 0)

            @pl.when(n_sel % 2 == 1)
            def _():
                scan_chunk(n_sel - 1, off_fin)
            pltpu.async_copy(candv_v[b], cv_hbm.at[row], sem_o[b])
            pltpu.async_copy(candi_v[b], ci_hbm.at[row], sem_o[b])

        loads_start(row0, 0)
        loads_start(row0 + 1, 1)

        def pair_body(g, n_sel_b):
            a = row0 + 2 * g
            loads_wait(0)
            n_sel_a = compact(a, 0)

            @pl.when(g < npairs - 1)
            def _():
                loads_start(a + 2, 0)

            gather_start(0)

            @pl.when(g > 1)
            def _():
                out_wait(1)

            @pl.when(g > 0)
            def _():
                gather_wait(1)
                scan_row(a - 1, n_sel_b, 1)

            loads_wait(1)
            n_sel_b2 = compact(a + 1, 1)

            @pl.when(g < npairs - 1)
            def _():
                loads_start(a + 3, 1)

            gather_start(1)

            gather_wait(0)

            @pl.when(g > 0)
            def _():
                out_wait(0)

            scan_row(a, n_sel_a, 0)
            return n_sel_b2

        n_sel_last = lax.fori_loop(0, npairs, pair_body, 0)
        gather_wait(1)
        out_wait(1)
        scan_row(row0 + rows_per_w - 1, n_sel_last, 1)
        out_wait(0)
        out_wait(1)

    return k3(scores2d, cmax, tb)


def _topk_body(cv_ref, ci_ref, os_ref, oi_ref, work):
    work[...] = cv_ref[...]
    idx = ci_ref[...]
    lane = lax.broadcasted_iota(jnp.int32, (NQ, 128), 1)
    os_ref[...] = jnp.zeros((NQ, 128), jnp.float32)
    oi_ref[...] = jnp.zeros((NQ, 128), jnp.int32)

    def body(it, carry):
        v = work[...]
        m = jnp.max(v, axis=1, keepdims=True)
        is_m = v == m
        sel = jnp.min(jnp.where(is_m, idx, BIG), axis=1, keepdims=True)
        hit = lane == it
        os_ref[...] = jnp.where(hit, m, os_ref[...])
        oi_ref[...] = jnp.where(hit, sel, oi_ref[...])
        work[...] = jnp.where(is_m & (idx == sel), NEG, v)
        return carry

    lax.fori_loop(0, KTOP, body, 0)


def _topk(cv, ci):
    return pl.pallas_call(
        _topk_body,
        in_specs=[
            pl.BlockSpec((NQ, CAND), lambda: (0, 0)),
            pl.BlockSpec((NQ, CAND), lambda: (0, 0)),
        ],
        out_specs=[
            pl.BlockSpec((NQ, 128), lambda: (0, 0)),
            pl.BlockSpec((NQ, 128), lambda: (0, 0)),
        ],
        out_shape=[
            jax.ShapeDtypeStruct((NQ, 128), jnp.float32),
            jax.ShapeDtypeStruct((NQ, 128), jnp.int32),
        ],
        scratch_shapes=[pltpu.VMEM((NQ, CAND), jnp.float32)],
    )(cv, ci)


def kernel(query, vectors, k):
    del k
    vpad = jnp.concatenate(
        [vectors, jnp.zeros((NPAD - NV, D), jnp.float32)], axis=0)
    scores, cmax3 = _scores_and_chunkmax(query, vpad)
    cmax = cmax3.transpose(1, 0, 2).reshape(NQ, NCHUNK)
    tb = _threshold(cmax)
    scores2d = scores.reshape(NQ * NCHUNK, 128)
    cv, ci = _sc_select(scores2d, cmax, tb)
    os_, oi_ = _topk(cv, ci)
    return (os_[:, :KTOP], oi_[:, :KTOP])
